# baseline (speedup 1.0000x reference)
import functools

import jax
import jax.numpy as jnp
from jax import lax
from jax.experimental import pallas as pl
from jax.experimental.pallas import tpu as pltpu
from jax.experimental.pallas import tpu_sc as plsc

N_EMB = 1000000
DIM = 64
NCB = 8
CB_SIZE = 256
CB_DIM = 8
BATCH = 4096
HIST = 200
N = BATCH * HIST

NW = 32
PER_W = N // NW
SUB = 128
CHUNK = 1024
HALF = CHUNK // 2
NSUB = CHUNK // SUB
NCHUNK = PER_W // CHUNK
G2 = HALF * NCB // SUB

STRIPE = 4000
NSTRIPE = N_EMB // STRIPE
TPW = (NSTRIPE + NW - 1) // NW

_SC_PARAMS = pltpu.CompilerParams(
    needs_layout_passes=False, use_tc_tiling_on_sc=False)


def _build_prep_kernel():
    mesh = plsc.VectorSubcoreMesh(core_axis_name="c", subcore_axis_name="s")

    @functools.partial(
        pl.kernel,
        out_type=jax.ShapeDtypeStruct((N_EMB * NCB,), jnp.int32),
        mesh=mesh,
        scratch_types=[
            pltpu.VMEM((2, NCB, STRIPE), jnp.int32),
            pltpu.VMEM((2, STRIPE * NCB), jnp.int32),
            pltpu.SemaphoreType.DMA,
            pltpu.SemaphoreType.DMA,
            pltpu.SemaphoreType.DMA,
        ],
        compiler_params=_SC_PARAMS,
    )
    def prep_sc(codes_hbm, out_hbm, in_v, out_v, sem_i, sem_s0, sem_s1):
        wid = lax.axis_index("c") * 16 + lax.axis_index("s")
        sem_s = (sem_s0, sem_s1)
        iota16 = lax.iota(jnp.int32, 16)

        def stripe_of(t):
            return t * NW + wid

        def fetch(t, slot):
            s = stripe_of(t)

            @pl.when(s < NSTRIPE)
            def _():
                off = pl.multiple_of(s * STRIPE, 8)
                pltpu.async_copy(
                    codes_hbm.at[:, pl.ds(off, STRIPE)],
                    in_v.at[slot], sem_i)

        fetch(0, 0)
        for t in range(TPW):
            p = t % 2
            s = stripe_of(t)
            if t + 1 < TPW:
                fetch(t + 1, 1 - p)

            @pl.when(s < NSTRIPE)
            def _():
                pltpu.make_async_copy(
                    codes_hbm.at[:, pl.ds(0, STRIPE)],
                    in_v.at[p], sem_i).wait()
                if t >= 2:
                    pltpu.make_async_copy(
                        out_hbm.at[pl.ds(0, STRIPE * NCB)],
                        out_v.at[p], sem_s[p]).wait()

                @pl.loop(0, STRIPE // 16)
                def grp(i):
                    v0 = i * 16
                    ovec = (v0 + iota16) * NCB
                    vals = [in_v[p, h, pl.ds(v0, 16)] + (h * CB_SIZE)
                            for h in range(NCB)]
                    for h in range(NCB):
                        plsc.store_scatter(out_v.at[p], [ovec + h], vals[h])

                off = pl.multiple_of(s * (STRIPE * NCB), 8)
                pltpu.async_copy(
                    out_v.at[p],
                    out_hbm.at[pl.ds(off, STRIPE * NCB)], sem_s[p])

        for t in range(max(TPW - 2, 0), TPW):
            p = t % 2
            s = stripe_of(t)

            @pl.when(s < NSTRIPE)
            def _():
                pltpu.make_async_copy(
                    out_hbm.at[pl.ds(0, STRIPE * NCB)],
                    out_v.at[p], sem_s[p]).wait()

    return prep_sc


def _build_main_kernel():
    mesh = plsc.VectorSubcoreMesh(core_axis_name="c", subcore_axis_name="s")

    @functools.partial(
        pl.kernel,
        out_type=jax.ShapeDtypeStruct((N * NCB, CB_DIM), jnp.float32),
        mesh=mesh,
        scratch_types=[
            pltpu.VMEM((2, NSUB, SUB), jnp.int32),
            pltpu.VMEM((2, CHUNK, NCB), jnp.int32),
            pltpu.VMEM((CHUNK * NCB,), jnp.int32),
            pltpu.VMEM((2, HALF * NCB, CB_DIM), jnp.float32),
            pltpu.VMEM_SHARED((NCB * CB_SIZE, CB_DIM), jnp.float32),
            pltpu.SemaphoreType.DMA,
            pltpu.SemaphoreType.DMA,
            pltpu.SemaphoreType.DMA,
            pltpu.SemaphoreType.DMA,
            pltpu.SemaphoreType.DMA,
        ],
        compiler_params=_SC_PARAMS,
    )
    def rpq_sc(idx_hbm, codes_plus_hbm, cb_hbm, out_hbm,
               idx_v, il_v, fl_v, out_v, cb_sh,
               sem_g, sem_c0, sem_c1, sem_o0, sem_o1):
        wid = lax.axis_index("c") * 16 + lax.axis_index("s")
        row_base = wid * PER_W
        sub_base = row_base // SUB
        sem_c = (sem_c0, sem_c1)
        sem_o = (sem_o0, sem_o1)

        @pl.when(lax.axis_index("s") == 0)
        def _():
            pltpu.sync_copy(cb_hbm, cb_sh)

        plsc.subcore_barrier()

        iota16 = lax.iota(jnp.int32, 16)

        def fetch(g, slot):
            sub_off = pl.multiple_of(sub_base + g * NSUB, 8)
            pltpu.sync_copy(idx_hbm.at[pl.ds(sub_off, NSUB)], idx_v.at[slot])
            for j in range(NSUB):
                pltpu.async_copy(codes_plus_hbm.at[idx_v.at[slot, j]],
                                 il_v.at[slot, pl.ds(j * SUB, SUB)],
                                 sem_g)

        fetch(0, 0)

        @pl.loop(0, NCHUNK)
        def chunk_loop(g):
            p = lax.rem(g, 2)
            pltpu.make_async_copy(codes_plus_hbm.at[pl.ds(0, CHUNK)],
                                  il_v.at[p], sem_g).wait()

            @pl.when(g + 1 < NCHUNK)
            def _():
                fetch(g + 1, 1 - p)

            pvec = jnp.broadcast_to(p, (16,))
            for k in range(2):
                @pl.when(g > 0)
                def _():
                    pltpu.make_async_copy(
                        out_hbm.at[pl.ds(0, HALF * NCB)],
                        out_v.at[k], sem_o[k]).wait()

                @pl.loop(0, HALF // 16)
                def row_loop(t):
                    rvec = k * HALF + t * 16 + iota16
                    rvec8 = rvec * NCB
                    vals = []
                    for h in range(NCB):
                        hvec = jnp.full((16,), h, jnp.int32)
                        vals.append(
                            plsc.load_gather(il_v, [pvec, rvec, hvec]))
                    for h in range(NCB):
                        plsc.store_scatter(fl_v, [rvec8 + h], vals[h])

                for j in range(G2):
                    pltpu.async_copy(
                        cb_sh.at[fl_v.at[pl.ds(k * HALF * NCB + j * SUB,
                                               SUB)]],
                        out_v.at[k, pl.ds(j * SUB, SUB)],
                        sem_c[k])

            for k in range(2):
                pltpu.make_async_copy(out_hbm.at[pl.ds(0, HALF * NCB)],
                                      out_v.at[k], sem_c[k]).wait()
                out_off = pl.multiple_of(
                    (row_base + g * CHUNK + k * HALF) * NCB, 8)
                pltpu.async_copy(out_v.at[k],
                                 out_hbm.at[pl.ds(out_off, HALF * NCB)],
                                 sem_o[k])

        for k in range(2):
            pltpu.make_async_copy(out_hbm.at[pl.ds(0, HALF * NCB)],
                                  out_v.at[k], sem_o[k]).wait()

    return rpq_sc


_PREP_SC = _build_prep_kernel()
_RPQ_SC = _build_main_kernel()


@jax.jit
def kernel(input, codes, codebooks):
    idx = input.reshape(N // SUB, SUB)
    codes_plus = _PREP_SC(codes).reshape(N_EMB, NCB)
    cb2 = codebooks.reshape(NCB * CB_SIZE, CB_DIM)
    out = _RPQ_SC(idx, codes_plus, cb2)
    return out.reshape(input.shape + (DIM,))
